# baseline (speedup 1.0000x reference)
import functools

import jax
import jax.numpy as jnp
from jax import lax
from jax.experimental import pallas as pl
from jax.experimental.pallas import tpu as pltpu
from jax.experimental.pallas import tpu_sc as plsc

_NUM_CORES = 2
_NUM_SUBCORES = 16
_NUM_WORKERS = _NUM_CORES * _NUM_SUBCORES


def _sc_pool(x, lengths, table):
    B, S = x.shape
    _, D = table.shape
    rpw = B // _NUM_WORKERS
    nvec = D // 16
    chunks = []
    off = 0
    while off < S:
        sz = min(128, S - off)
        chunks.append((off, sz))
        off += sz

    mesh = plsc.VectorSubcoreMesh(core_axis_name="c", subcore_axis_name="s")

    @functools.partial(
        pl.kernel,
        mesh=mesh,
        out_type=jax.ShapeDtypeStruct((B, D), jnp.float32),
        compiler_params=pltpu.CompilerParams(use_tc_tiling_on_sc=False),
        scratch_types=[
            pltpu.VMEM((rpw, S), jnp.int32),
            pltpu.VMEM((rpw + 16,), jnp.int32),
            pltpu.VMEM((S, D), jnp.float32),
            pltpu.VMEM((rpw, D), jnp.float32),
            pltpu.SemaphoreType.DMA,
        ],
    )
    def k(x_hbm, len_hbm, table_hbm, out_hbm, xv, lenv, rowsv, repv, gsem):
        wid = lax.axis_index("s") * _NUM_CORES + lax.axis_index("c")
        base = wid * rpw
        pltpu.sync_copy(x_hbm.at[pl.ds(base, rpw)], xv)
        pltpu.sync_copy(len_hbm.at[pl.ds(base, rpw)], lenv.at[pl.ds(0, rpw)])

        def row_body(r, carry):
            handles = []
            for off, sz in chunks:
                handles.append(
                    pltpu.async_copy(
                        table_hbm.at[xv.at[r, pl.ds(off, sz)]],
                        rowsv.at[pl.ds(off, sz)],
                        gsem,
                    )
                )
            for h in handles:
                h.wait()
            l = lenv[pl.ds(r, 16)][0]

            def acc_body(j, accs):
                return tuple(
                    accs[q] + rowsv[j, pl.ds(q * 16, 16)] for q in range(nvec)
                )

            accs = tuple(jnp.zeros((16,), jnp.float32) for _ in range(nvec))
            accs = lax.fori_loop(0, l, acc_body, accs)
            lvec = jnp.full((16,), l, jnp.float32)
            inv = 1.0 / lvec
            for q in range(nvec):
                repv[r, pl.ds(q * 16, 16)] = accs[q] * inv
            return carry

        lax.fori_loop(0, rpw, row_body, 0)
        pltpu.sync_copy(repv, out_hbm.at[pl.ds(base, rpw)])

    return k(x, lengths, table)


def _tc_head(rep, W, b2):
    B, _ = rep.shape
    C = W.shape[1]

    def body(rep_ref, w_ref, b_ref, o_ref):
        r = jnp.maximum(rep_ref[...], 0.0)
        o_ref[...] = (
            lax.dot_general(
                r, w_ref[...], (((1,), (0,)), ((), ())),
                preferred_element_type=jnp.float32,
            )
            + b_ref[...]
        )

    return pl.pallas_call(
        body,
        out_shape=jax.ShapeDtypeStruct((B, C), jnp.float32),
    )(rep, W, b2)


def kernel(x, lengths, table, W, b):
    x = x.astype(jnp.int32)
    lengths = lengths.astype(jnp.int32)
    rep = _sc_pool(x, lengths, table)
    return _tc_head(rep, W, b.reshape(1, -1))


# speedup vs baseline: 1.0741x; 1.0741x over previous
import functools

import jax
import jax.numpy as jnp
from jax import lax
from jax.experimental import pallas as pl
from jax.experimental.pallas import tpu as pltpu
from jax.experimental.pallas import tpu_sc as plsc

_NUM_CORES = 2
_NUM_SUBCORES = 16
_NUM_WORKERS = _NUM_CORES * _NUM_SUBCORES


def _sc_pool(x, lengths, table):
    B, S = x.shape
    _, D = table.shape
    rpw = B // _NUM_WORKERS
    nvec = D // 16
    ch = 40
    nch_max = S // ch
    nbuf = 4

    mesh = plsc.VectorSubcoreMesh(core_axis_name="c", subcore_axis_name="s")

    @functools.partial(
        pl.kernel,
        mesh=mesh,
        out_type=jax.ShapeDtypeStruct((B, D), jnp.float32),
        compiler_params=pltpu.CompilerParams(use_tc_tiling_on_sc=False),
        scratch_types=[
            pltpu.VMEM((rpw, S), jnp.int32),
            pltpu.VMEM((rpw + 16,), jnp.int32),
            pltpu.VMEM((nbuf, S, D), jnp.float32),
            pltpu.VMEM((rpw, D), jnp.float32),
        ]
        + [pltpu.SemaphoreType.DMA] * nbuf,
    )
    def k(x_hbm, len_hbm, table_hbm, out_hbm, xv, lenv, rowsv, repv, *sems):
        wid = lax.axis_index("s") * _NUM_CORES + lax.axis_index("c")
        base = wid * rpw
        pltpu.sync_copy(x_hbm.at[pl.ds(base, rpw)], xv)
        pltpu.sync_copy(len_hbm.at[pl.ds(base, rpw)], lenv.at[pl.ds(0, rpw)])

        def nchunks(r):
            l = lenv[pl.ds(r, 16)][0]
            return l, (l + (ch - 1)) // ch

        def fire(r, k_buf):
            _, nch = nchunks(r)
            for c in range(nch_max):

                @pl.when(c < nch)
                def _():
                    pltpu.async_copy(
                        table_hbm.at[xv.at[r, pl.ds(c * ch, ch)]],
                        rowsv.at[k_buf, pl.ds(c * ch, ch)],
                        sems[k_buf],
                    )

        def drain(r, k_buf):
            _, nch = nchunks(r)
            for c in range(nch_max):

                @pl.when(c < nch)
                def _():
                    pltpu.make_async_copy(
                        table_hbm.at[pl.ds(0, ch)],
                        rowsv.at[k_buf, pl.ds(c * ch, ch)],
                        sems[k_buf],
                    ).wait()

        def accumulate(r, k_buf):
            l, nch = nchunks(r)

            def chunk_body(c, accs):
                j0 = c * ch
                for jj in range(ch):
                    j = j0 + jj
                    take = j < l
                    accs = tuple(
                        accs[q]
                        + jnp.where(take, rowsv[k_buf, j, pl.ds(q * 16, 16)], 0.0)
                        for q in range(nvec)
                    )
                return accs

            accs = tuple(jnp.zeros((16,), jnp.float32) for _ in range(nvec))
            accs = lax.fori_loop(0, nch, chunk_body, accs)
            inv = 1.0 / jnp.full((16,), l, jnp.float32)
            for q in range(nvec):
                repv[r, pl.ds(q * 16, 16)] = accs[q] * inv

        for k_buf in range(nbuf):
            fire(k_buf, k_buf)

        def body(i, carry):
            for k_buf in range(nbuf):
                r = i * nbuf + k_buf
                drain(r, k_buf)
                accumulate(r, k_buf)
                nxt = r + nbuf

                @pl.when(nxt < rpw)
                def _():
                    fire(nxt, k_buf)

            return carry

        lax.fori_loop(0, rpw // nbuf, body, 0)
        pltpu.sync_copy(repv, out_hbm.at[pl.ds(base, rpw)])

    return k(x, lengths, table)


def _tc_head(rep, W, b2):
    B, _ = rep.shape
    C = W.shape[1]

    def body(rep_ref, w_ref, b_ref, o_ref):
        r = jnp.maximum(rep_ref[...], 0.0)
        o_ref[...] = (
            lax.dot_general(
                r, w_ref[...], (((1,), (0,)), ((), ())),
                preferred_element_type=jnp.float32,
            )
            + b_ref[...]
        )

    return pl.pallas_call(
        body,
        out_shape=jax.ShapeDtypeStruct((B, C), jnp.float32),
    )(rep, W, b2)


def kernel(x, lengths, table, W, b):
    x = x.astype(jnp.int32)
    lengths = lengths.astype(jnp.int32)
    rep = _sc_pool(x, lengths, table)
    return _tc_head(rep, W, b.reshape(1, -1))
